# Initial kernel scaffold; baseline (speedup 1.0000x reference)
import jax
import jax.numpy as jnp
from jax.experimental import pallas as pl


def kernel(x, edge_index, batch, W1, b1, W2, b2, Wres, bres, Wa1, ba1, Wa2, ba2):
    raise NotImplementedError("write your pallas kernel here")


# speedup vs baseline: 5.8846x; 5.8846x over previous
import functools

import jax
import jax.numpy as jnp
from jax import lax
from jax.experimental import pallas as pl
from jax.experimental.pallas import tpu as pltpu
from jax.experimental.pallas import tpu_sc as plsc

N = 10000
E = 320000
K = 64
RR = 5120
EPAD = RR * K - E
CH = RR // 16
NPAD = 10240
HN = NPAD // 2
ACC = HN
ZB1 = N
T2H = N + 1000
HPT = HN // 16
HACC = NPAD + 128
NPT = NPAD // 16
BN = 1000
NB = N // BN

_f32 = jnp.float32


def _sc_mesh():
    return plsc.VectorSubcoreMesh(core_axis_name="c", subcore_axis_name="s")


def _agg_phase(tblr, srcr, dstr, zr, srcv, dstv, rows, acc, sem, c, s,
               src_row0, zbase):
    lo = c * HN
    pltpu.sync_copy(zr.at[pl.ds(s * HPT, HPT)], acc.at[pl.ds(s * HPT, HPT)])
    pltpu.sync_copy(srcr.at[pl.ds(src_row0 + s * CH, CH)], srcv)
    pltpu.sync_copy(dstr.at[pl.ds(s * CH, CH)], dstv)
    plsc.subcore_barrier()

    def body(j, carry):
        for cc in range(0, K, 16):
            d = dstv[j, pl.ds(cc, 16)]
            inr = jnp.logical_and(d >= lo, d < lo + HN)
            dstv[j, pl.ds(cc, 16)] = jnp.where(inr, d - lo, d & 4095)
            sv = srcv[j, pl.ds(cc, 16)]
            srcv[j, pl.ds(cc, 16)] = jnp.where(inr, sv, zbase + (sv & 511))
        pltpu.async_copy(tblr.at[srcv.at[j]], rows, sem).wait()
        pltpu.sync_copy(rows, acc.at[dstv.at[j]], add=True)
        return carry

    lax.fori_loop(0, CH, body, 0)
    plsc.subcore_barrier()


_AGG_SCRATCH = [
    pltpu.VMEM((CH, K), jnp.int32),
    pltpu.VMEM((CH, K), jnp.int32),
    pltpu.VMEM((K, 128), _f32),
    pltpu.VMEM_SHARED((ACC, 128), _f32),
    pltpu.SemaphoreType.DMA,
]


def _sc_agg1(tbl, srcarr, dstp, ztbl):
    @functools.partial(
        pl.kernel,
        out_type=jax.ShapeDtypeStruct((NPAD, 128), _f32),
        mesh=_sc_mesh(),
        scratch_types=list(_AGG_SCRATCH),
    )
    def k(tblr, srcr, dstr, zr, out, srcv, dstv, rows, acc, sem):
        c = lax.axis_index("c")
        s = lax.axis_index("s")
        _agg_phase(tblr, srcr, dstr, zr, srcv, dstv, rows, acc, sem, c, s,
                   0, ZB1)
        pltpu.sync_copy(acc.at[pl.ds(s * HPT, HPT)],
                        out.at[pl.ds(c * HN + s * HPT, HPT)])

    return k(tbl, srcarr, dstp, ztbl)


def _sc_agg2(tbl, src2, dstp, ztbl):
    @functools.partial(
        pl.kernel,
        out_type=jax.ShapeDtypeStruct((2 * NPAD, 128), _f32),
        mesh=_sc_mesh(),
        scratch_types=list(_AGG_SCRATCH),
    )
    def k(tblr, srcr, dstr, zr, out, srcv, dstv, rows, acc, sem):
        c = lax.axis_index("c")
        s = lax.axis_index("s")
        for p in range(2):
            _agg_phase(tblr, srcr, dstr, zr, srcv, dstv, rows, acc, sem,
                       c, s, p * RR, N + p * T2H)
            pltpu.sync_copy(acc.at[pl.ds(s * HPT, HPT)],
                            out.at[pl.ds(p * NPAD + c * HN + s * HPT, HPT)])
            plsc.subcore_barrier()

    return k(tbl, src2, dstp, ztbl)


def _dinv_block(dp_ref):
    deg = dp_ref[:, 0:1] + 1.0
    return lax.rsqrt(jnp.maximum(deg, 1e-12))


def _tc1(x, W1, degp3):
    def body(x_ref, w_ref, dp_ref, y_ref):
        i = pl.program_id(0)

        @pl.when(i < NB)
        def _():
            dinv = _dinv_block(dp_ref)
            y_ref[...] = jnp.dot(x_ref[...], w_ref[...],
                                 preferred_element_type=_f32) * dinv

        @pl.when(i == NB)
        def _():
            y_ref[...] = jnp.zeros_like(y_ref)

    clamp = lambda i: (jnp.minimum(i, NB - 1), 0)
    return pl.pallas_call(
        body,
        grid=(NB + 1,),
        in_specs=[
            pl.BlockSpec((BN, 128), clamp),
            pl.BlockSpec((128, 128), lambda i: (0, 0)),
            pl.BlockSpec((BN, 128), clamp),
        ],
        out_specs=pl.BlockSpec((BN, 128), lambda i: (i, 0)),
        out_shape=jax.ShapeDtypeStruct((N + 1000, 128), _f32),
    )(x, W1, degp3)


def _tc2(s1, y1, degp3, b1r, W2, Wres, bresr):
    def body(s_ref, y_ref, dp_ref, b1_ref, w2_ref, wr_ref, br_ref, y2_ref, r_ref):
        i = pl.program_id(0)

        @pl.when(i < NB)
        def _():
            dinv = _dinv_block(dp_ref)
            h1 = jax.nn.relu((s_ref[...] + y_ref[...]) * dinv + b1_ref[...])
            m2 = jnp.dot(h1, w2_ref[...], preferred_element_type=_f32) * dinv
            y2_ref[0, :, :] = m2[:, :128]
            y2_ref[1, :, :] = m2[:, 128:]
            r_ref[...] = jnp.dot(h1, wr_ref[...],
                                 preferred_element_type=_f32) + br_ref[...]

        @pl.when(i == NB)
        def _():
            y2_ref[...] = jnp.zeros_like(y2_ref)

    clamp = lambda i: (jnp.minimum(i, NB - 1), 0)
    return pl.pallas_call(
        body,
        grid=(NB + 1,),
        in_specs=[
            pl.BlockSpec((BN, 128), clamp),
            pl.BlockSpec((BN, 128), clamp),
            pl.BlockSpec((BN, 128), clamp),
            pl.BlockSpec((1, 128), lambda i: (0, 0)),
            pl.BlockSpec((128, 256), lambda i: (0, 0)),
            pl.BlockSpec((128, 256), lambda i: (0, 0)),
            pl.BlockSpec((1, 256), lambda i: (0, 0)),
        ],
        out_specs=[
            pl.BlockSpec((2, BN, 128), lambda i: (0, i, 0)),
            pl.BlockSpec((BN, 256), clamp),
        ],
        out_shape=[
            jax.ShapeDtypeStruct((2, T2H, 128), _f32),
            jax.ShapeDtypeStruct((N, 256), _f32),
        ],
    )(s1, y1, degp3, b1r, W2, Wres, bresr)


def _tc3(s2_3, y2s, r, degp3, b2r, Wa1, ba1r, wa2r, ba2r, batch3):
    def body(s_ref, y_ref, r_ref, dp_ref, b2_ref, wa1_ref, ba1_ref,
             wa2_ref, ba2_ref, b_ref, out_ref, num_acc, cnt_acc):
        i = pl.program_id(0)
        dinv = _dinv_block(dp_ref)
        pre = jnp.concatenate([s_ref[0] + y_ref[0], s_ref[1] + y_ref[1]],
                              axis=1) * dinv
        h2 = jax.nn.relu(pre + b2_ref[...] + r_ref[...])
        t = jax.nn.relu(jnp.dot(h2, wa1_ref[...], preferred_element_type=_f32)
                        + ba1_ref[...])
        wl = jnp.sum(t * wa2_ref[...], axis=1, keepdims=True) + ba2_ref[...]
        w = jax.nn.sigmoid(wl)
        hw = h2 * w
        bb = b_ref[0]
        oneh = (bb == lax.broadcasted_iota(jnp.int32, (64, BN), 0)).astype(_f32)

        @pl.when(i == 0)
        def _():
            num_acc[...] = jnp.zeros_like(num_acc)
            cnt_acc[...] = jnp.zeros_like(cnt_acc)

        num_acc[...] = num_acc[...] + jnp.dot(oneh, hw,
                                              preferred_element_type=_f32)
        cnt_acc[...] = cnt_acc[...] + jnp.sum(oneh, axis=1, keepdims=True)

        @pl.when(i == NB - 1)
        def _():
            emb = num_acc[...] / jnp.maximum(cnt_acc[:, 0:1], 1.0)
            nrm2 = jnp.sum(emb * emb, axis=1, keepdims=True)
            out_ref[...] = emb / jnp.maximum(jnp.sqrt(nrm2), 1e-12)

    return pl.pallas_call(
        body,
        grid=(NB,),
        in_specs=[
            pl.BlockSpec((2, BN, 128), lambda i: (0, i, 0)),
            pl.BlockSpec((2, BN, 128), lambda i: (0, i, 0)),
            pl.BlockSpec((BN, 256), lambda i: (i, 0)),
            pl.BlockSpec((BN, 128), lambda i: (i, 0)),
            pl.BlockSpec((1, 256), lambda i: (0, 0)),
            pl.BlockSpec((256, 64), lambda i: (0, 0)),
            pl.BlockSpec((1, 64), lambda i: (0, 0)),
            pl.BlockSpec((1, 64), lambda i: (0, 0)),
            pl.BlockSpec((1, 1), lambda i: (0, 0)),
            pl.BlockSpec((1, 1, BN), lambda i: (i, 0, 0)),
        ],
        out_specs=pl.BlockSpec((64, 256), lambda i: (0, 0)),
        out_shape=jax.ShapeDtypeStruct((64, 256), _f32),
        scratch_shapes=[
            pltpu.VMEM((64, 256), _f32),
            pltpu.VMEM((64, 128), _f32),
        ],
    )(s2_3, y2s, r, degp3, b2r, Wa1, ba1r, wa2r, ba2r, batch3)


def kernel(x, edge_index, batch, W1, b1, W2, b2, Wres, bres, Wa1, ba1, Wa2, ba2):
    pad_i = jnp.arange(EPAD, dtype=jnp.int32)
    srcp = jnp.concatenate([edge_index[0], pad_i % N]).reshape(RR, K)
    dstp = jnp.concatenate([edge_index[1], NPAD + (pad_i % 64)]).reshape(RR, K)

    z128 = jnp.zeros((ACC, 128), _f32)

    ones_tbl = jnp.concatenate([jnp.ones((N, 128), _f32),
                                jnp.zeros((1000, 128), _f32)], axis=0)
    degp3 = _sc_agg1(ones_tbl, srcp, dstp, z128)

    y1 = _tc1(x, W1, degp3)
    s1 = _sc_agg1(y1, srcp, dstp, z128)
    y2s, r = _tc2(s1, y1, degp3, b1.reshape(1, 128),
                  W2, Wres, bres.reshape(1, 256))
    y2t = y2s.reshape(2 * T2H, 128)
    s2 = _sc_agg2(y2t, jnp.concatenate([srcp, srcp + T2H], axis=0), dstp, z128)
    return _tc3(s2.reshape(2, NPAD, 128), y2s, r, degp3, b2.reshape(1, 256),
                Wa1, ba1.reshape(1, 64), Wa2.reshape(1, 64),
                ba2.reshape(1, 1), batch.reshape(NB, 1, BN))


# speedup vs baseline: 7.4301x; 1.2626x over previous
import functools

import jax
import jax.numpy as jnp
from jax import lax
from jax.experimental import pallas as pl
from jax.experimental.pallas import tpu as pltpu
from jax.experimental.pallas import tpu_sc as plsc

N = 10000
E = 320000
K = 64
RR = 5120
EPAD = RR * K - E
CH = RR // 16
CH2 = CH // 2
NPAD = 10240
HN = NPAD // 2
ACC = HN
ZB1 = N
T2H = N + 1000
HPT = HN // 16
HACC = NPAD + 128
NPT = NPAD // 16
BN = 1000
NB = N // BN

_f32 = jnp.float32


def _sc_mesh():
    return plsc.VectorSubcoreMesh(core_axis_name="c", subcore_axis_name="s")


def _agg_phase(tblr, srcr, dstr, zr, srcv, dstv, rows, acc, sem, sem2, c, s,
               src_row0, zbase):
    lo = c * HN
    pltpu.sync_copy(zr.at[pl.ds(s * HPT, HPT)], acc.at[pl.ds(s * HPT, HPT)])
    plsc.subcore_barrier()

    def adjust(j):
        for cc in range(0, K, 16):
            d = dstv[j, pl.ds(cc, 16)]
            inr = jnp.logical_and(d >= lo, d < lo + HN)
            dstv[j, pl.ds(cc, 16)] = jnp.where(inr, d - lo, d & 4095)
            sv = srcv[j, pl.ds(cc, 16)]
            srcv[j, pl.ds(cc, 16)] = jnp.where(inr, sv, zbase + (sv & 511))

    for hb in range(CH // CH2):
        pltpu.sync_copy(srcr.at[pl.ds(src_row0 + s * CH + hb * CH2, CH2)], srcv)
        pltpu.sync_copy(dstr.at[pl.ds(s * CH + hb * CH2, CH2)], dstv)
        adjust(0)
        pltpu.async_copy(tblr.at[srcv.at[0]], rows[0], sem).wait()

        def body(j2, carry):
            j = 2 * j2
            for par in range(2):
                jn = jnp.minimum(j + par + 1, CH2 - 1)
                hs = pltpu.async_copy(rows[par], acc.at[dstv.at[j + par]],
                                      sem2, add=True)
                adjust(jn)
                hg = pltpu.async_copy(tblr.at[srcv.at[jn]], rows[1 - par], sem)
                hg.wait()
                hs.wait()
            return carry

        lax.fori_loop(0, CH2 // 2, body, 0)
    plsc.subcore_barrier()


_AGG_SCRATCH = [
    pltpu.VMEM((CH2, K), jnp.int32),
    pltpu.VMEM((CH2, K), jnp.int32),
    pltpu.VMEM((K, 128), _f32),
    pltpu.VMEM((K, 128), _f32),
    pltpu.VMEM_SHARED((ACC, 128), _f32),
    pltpu.SemaphoreType.DMA,
    pltpu.SemaphoreType.DMA,
]


def _sc_agg1(tbl, srcarr, dstp, ztbl):
    @functools.partial(
        pl.kernel,
        out_type=jax.ShapeDtypeStruct((NPAD, 128), _f32),
        mesh=_sc_mesh(),
        scratch_types=list(_AGG_SCRATCH),
    )
    def k(tblr, srcr, dstr, zr, out, srcv, dstv, rows0, rows1, acc, sem, sem2):
        c = lax.axis_index("c")
        s = lax.axis_index("s")
        _agg_phase(tblr, srcr, dstr, zr, srcv, dstv, (rows0, rows1), acc,
                   sem, sem2, c, s, 0, ZB1)
        pltpu.sync_copy(acc.at[pl.ds(s * HPT, HPT)],
                        out.at[pl.ds(c * HN + s * HPT, HPT)])

    return k(tbl, srcarr, dstp, ztbl)


def _sc_agg2(tbl, src2, dstp, ztbl):
    @functools.partial(
        pl.kernel,
        out_type=jax.ShapeDtypeStruct((2 * NPAD, 128), _f32),
        mesh=_sc_mesh(),
        scratch_types=list(_AGG_SCRATCH),
    )
    def k(tblr, srcr, dstr, zr, out, srcv, dstv, rows0, rows1, acc, sem, sem2):
        c = lax.axis_index("c")
        s = lax.axis_index("s")
        for p in range(2):
            _agg_phase(tblr, srcr, dstr, zr, srcv, dstv, (rows0, rows1), acc,
                       sem, sem2, c, s, p * RR, N + p * T2H)
            pltpu.sync_copy(acc.at[pl.ds(s * HPT, HPT)],
                            out.at[pl.ds(p * NPAD + c * HN + s * HPT, HPT)])
            plsc.subcore_barrier()

    return k(tbl, src2, dstp, ztbl)


def _dinv_block(dp_ref):
    deg = dp_ref[:, 0:1] + 1.0
    return lax.rsqrt(jnp.maximum(deg, 1e-12))


def _tc1(x, W1, degp3):
    def body(x_ref, w_ref, dp_ref, y_ref):
        i = pl.program_id(0)

        @pl.when(i < NB)
        def _():
            dinv = _dinv_block(dp_ref)
            y_ref[...] = jnp.dot(x_ref[...], w_ref[...],
                                 preferred_element_type=_f32) * dinv

        @pl.when(i == NB)
        def _():
            y_ref[...] = jnp.zeros_like(y_ref)

    clamp = lambda i: (jnp.minimum(i, NB - 1), 0)
    return pl.pallas_call(
        body,
        grid=(NB + 1,),
        in_specs=[
            pl.BlockSpec((BN, 128), clamp),
            pl.BlockSpec((128, 128), lambda i: (0, 0)),
            pl.BlockSpec((BN, 128), clamp),
        ],
        out_specs=pl.BlockSpec((BN, 128), lambda i: (i, 0)),
        out_shape=jax.ShapeDtypeStruct((N + 1000, 128), _f32),
    )(x, W1, degp3)


def _tc2(s1, y1, degp3, b1r, W2, Wres, bresr):
    def body(s_ref, y_ref, dp_ref, b1_ref, w2_ref, wr_ref, br_ref, y2_ref, r_ref):
        i = pl.program_id(0)

        @pl.when(i < NB)
        def _():
            dinv = _dinv_block(dp_ref)
            h1 = jax.nn.relu((s_ref[...] + y_ref[...]) * dinv + b1_ref[...])
            m2 = jnp.dot(h1, w2_ref[...], preferred_element_type=_f32) * dinv
            y2_ref[0, :, :] = m2[:, :128]
            y2_ref[1, :, :] = m2[:, 128:]
            r_ref[...] = jnp.dot(h1, wr_ref[...],
                                 preferred_element_type=_f32) + br_ref[...]

        @pl.when(i == NB)
        def _():
            y2_ref[...] = jnp.zeros_like(y2_ref)

    clamp = lambda i: (jnp.minimum(i, NB - 1), 0)
    return pl.pallas_call(
        body,
        grid=(NB + 1,),
        in_specs=[
            pl.BlockSpec((BN, 128), clamp),
            pl.BlockSpec((BN, 128), clamp),
            pl.BlockSpec((BN, 128), clamp),
            pl.BlockSpec((1, 128), lambda i: (0, 0)),
            pl.BlockSpec((128, 256), lambda i: (0, 0)),
            pl.BlockSpec((128, 256), lambda i: (0, 0)),
            pl.BlockSpec((1, 256), lambda i: (0, 0)),
        ],
        out_specs=[
            pl.BlockSpec((2, BN, 128), lambda i: (0, i, 0)),
            pl.BlockSpec((BN, 256), clamp),
        ],
        out_shape=[
            jax.ShapeDtypeStruct((2, T2H, 128), _f32),
            jax.ShapeDtypeStruct((N, 256), _f32),
        ],
    )(s1, y1, degp3, b1r, W2, Wres, bresr)


def _tc3(s2_3, y2s, r, degp3, b2r, Wa1, ba1r, wa2r, ba2r, batch3):
    def body(s_ref, y_ref, r_ref, dp_ref, b2_ref, wa1_ref, ba1_ref,
             wa2_ref, ba2_ref, b_ref, out_ref, num_acc, cnt_acc):
        i = pl.program_id(0)
        dinv = _dinv_block(dp_ref)
        pre = jnp.concatenate([s_ref[0] + y_ref[0], s_ref[1] + y_ref[1]],
                              axis=1) * dinv
        h2 = jax.nn.relu(pre + b2_ref[...] + r_ref[...])
        t = jax.nn.relu(jnp.dot(h2, wa1_ref[...], preferred_element_type=_f32)
                        + ba1_ref[...])
        wl = jnp.sum(t * wa2_ref[...], axis=1, keepdims=True) + ba2_ref[...]
        w = jax.nn.sigmoid(wl)
        hw = h2 * w
        bb = b_ref[0]
        oneh = (bb == lax.broadcasted_iota(jnp.int32, (64, BN), 0)).astype(_f32)

        @pl.when(i == 0)
        def _():
            num_acc[...] = jnp.zeros_like(num_acc)
            cnt_acc[...] = jnp.zeros_like(cnt_acc)

        num_acc[...] = num_acc[...] + jnp.dot(oneh, hw,
                                              preferred_element_type=_f32)
        cnt_acc[...] = cnt_acc[...] + jnp.sum(oneh, axis=1, keepdims=True)

        @pl.when(i == NB - 1)
        def _():
            emb = num_acc[...] / jnp.maximum(cnt_acc[:, 0:1], 1.0)
            nrm2 = jnp.sum(emb * emb, axis=1, keepdims=True)
            out_ref[...] = emb / jnp.maximum(jnp.sqrt(nrm2), 1e-12)

    return pl.pallas_call(
        body,
        grid=(NB,),
        in_specs=[
            pl.BlockSpec((2, BN, 128), lambda i: (0, i, 0)),
            pl.BlockSpec((2, BN, 128), lambda i: (0, i, 0)),
            pl.BlockSpec((BN, 256), lambda i: (i, 0)),
            pl.BlockSpec((BN, 128), lambda i: (i, 0)),
            pl.BlockSpec((1, 256), lambda i: (0, 0)),
            pl.BlockSpec((256, 64), lambda i: (0, 0)),
            pl.BlockSpec((1, 64), lambda i: (0, 0)),
            pl.BlockSpec((1, 64), lambda i: (0, 0)),
            pl.BlockSpec((1, 1), lambda i: (0, 0)),
            pl.BlockSpec((1, 1, BN), lambda i: (i, 0, 0)),
        ],
        out_specs=pl.BlockSpec((64, 256), lambda i: (0, 0)),
        out_shape=jax.ShapeDtypeStruct((64, 256), _f32),
        scratch_shapes=[
            pltpu.VMEM((64, 256), _f32),
            pltpu.VMEM((64, 128), _f32),
        ],
    )(s2_3, y2s, r, degp3, b2r, Wa1, ba1r, wa2r, ba2r, batch3)


def kernel(x, edge_index, batch, W1, b1, W2, b2, Wres, bres, Wa1, ba1, Wa2, ba2):
    pad_i = jnp.arange(EPAD, dtype=jnp.int32)
    srcp = jnp.concatenate([edge_index[0], pad_i % N]).reshape(RR, K)
    dstp = jnp.concatenate([edge_index[1], NPAD + (pad_i % 64)]).reshape(RR, K)

    z128 = jnp.zeros((ACC, 128), _f32)

    ones_tbl = jnp.concatenate([jnp.ones((N, 128), _f32),
                                jnp.zeros((1000, 128), _f32)], axis=0)
    degp3 = _sc_agg1(ones_tbl, srcp, dstp, z128)

    y1 = _tc1(x, W1, degp3)
    s1 = _sc_agg1(y1, srcp, dstp, z128)
    y2s, r = _tc2(s1, y1, degp3, b1.reshape(1, 128),
                  W2, Wres, bres.reshape(1, 256))
    y2t = y2s.reshape(2 * T2H, 128)
    s2 = _sc_agg2(y2t, jnp.concatenate([srcp, srcp + T2H], axis=0), dstp, z128)
    return _tc3(s2.reshape(2, NPAD, 128), y2s, r, degp3, b2.reshape(1, 256),
                Wa1, ba1.reshape(1, 64), Wa2.reshape(1, 64),
                ba2.reshape(1, 1), batch.reshape(NB, 1, BN))


# speedup vs baseline: 7.5913x; 1.0217x over previous
import functools

import jax
import jax.numpy as jnp
from jax import lax
from jax.experimental import pallas as pl
from jax.experimental.pallas import tpu as pltpu
from jax.experimental.pallas import tpu_sc as plsc

N = 10000
E = 320000
K = 64
RR = 5120
EPAD = RR * K - E
CH = RR // 16
CH2 = CH // 2
NPAD = 10240
HN = NPAD // 2
ACC = HN
ZB1 = N
T2H = N + 1000
HPT = HN // 16
HACC = NPAD + 128
NPT = NPAD // 16
BN = 1000
NB = N // BN

_f32 = jnp.float32


def _sc_mesh():
    return plsc.VectorSubcoreMesh(core_axis_name="c", subcore_axis_name="s")


def _agg_phase(tblr, srcr, dstr, zr, srcv, dstv, rows, acc, sem, sem2, c, s,
               src_row0, zbase):
    lo = c * HN
    pltpu.sync_copy(zr.at[pl.ds(s * HPT, HPT)], acc.at[pl.ds(s * HPT, HPT)])
    plsc.subcore_barrier()

    def adjust(j, carry):
        for cc in range(0, K, 16):
            d = dstv[j, pl.ds(cc, 16)]
            inr = jnp.logical_and(d >= lo, d < lo + HN)
            dstv[j, pl.ds(cc, 16)] = jnp.where(inr, d - lo, d & 4095)
            sv = srcv[j, pl.ds(cc, 16)]
            srcv[j, pl.ds(cc, 16)] = jnp.where(inr, sv, zbase + (sv & 511))
        return carry

    for hb in range(CH // CH2):
        pltpu.sync_copy(srcr.at[pl.ds(src_row0 + s * CH + hb * CH2, CH2)], srcv)
        pltpu.sync_copy(dstr.at[pl.ds(s * CH + hb * CH2, CH2)], dstv)
        lax.fori_loop(0, CH2, adjust, 0)
        pltpu.async_copy(tblr.at[srcv.at[0]], rows[0], sem).wait()

        def body(j2, carry):
            j = 2 * j2
            for par in range(2):
                jn = jnp.minimum(j + par + 1, CH2 - 1)
                hs = pltpu.async_copy(rows[par], acc.at[dstv.at[j + par]],
                                      sem2, add=True)
                hg = pltpu.async_copy(tblr.at[srcv.at[jn]], rows[1 - par], sem)
                hg.wait()
                hs.wait()
            return carry

        lax.fori_loop(0, CH2 // 2, body, 0)
    plsc.subcore_barrier()


_AGG_SCRATCH = [
    pltpu.VMEM((CH2, K), jnp.int32),
    pltpu.VMEM((CH2, K), jnp.int32),
    pltpu.VMEM((K, 128), _f32),
    pltpu.VMEM((K, 128), _f32),
    pltpu.VMEM_SHARED((ACC, 128), _f32),
    pltpu.SemaphoreType.DMA,
    pltpu.SemaphoreType.DMA,
]


def _sc_agg1(tbl, srcarr, dstp, ztbl):
    @functools.partial(
        pl.kernel,
        out_type=jax.ShapeDtypeStruct((NPAD, 128), _f32),
        mesh=_sc_mesh(),
        scratch_types=list(_AGG_SCRATCH),
    )
    def k(tblr, srcr, dstr, zr, out, srcv, dstv, rows0, rows1, acc, sem, sem2):
        c = lax.axis_index("c")
        s = lax.axis_index("s")
        _agg_phase(tblr, srcr, dstr, zr, srcv, dstv, (rows0, rows1), acc,
                   sem, sem2, c, s, 0, ZB1)
        pltpu.sync_copy(acc.at[pl.ds(s * HPT, HPT)],
                        out.at[pl.ds(c * HN + s * HPT, HPT)])

    return k(tbl, srcarr, dstp, ztbl)


def _sc_agg2(tbl, src2, dstp, ztbl):
    @functools.partial(
        pl.kernel,
        out_type=jax.ShapeDtypeStruct((2 * NPAD, 128), _f32),
        mesh=_sc_mesh(),
        scratch_types=list(_AGG_SCRATCH),
    )
    def k(tblr, srcr, dstr, zr, out, srcv, dstv, rows0, rows1, acc, sem, sem2):
        c = lax.axis_index("c")
        s = lax.axis_index("s")
        for p in range(2):
            _agg_phase(tblr, srcr, dstr, zr, srcv, dstv, (rows0, rows1), acc,
                       sem, sem2, c, s, p * RR, N + p * T2H)
            pltpu.sync_copy(acc.at[pl.ds(s * HPT, HPT)],
                            out.at[pl.ds(p * NPAD + c * HN + s * HPT, HPT)])
            plsc.subcore_barrier()

    return k(tbl, src2, dstp, ztbl)


def _dinv_block(dp_ref):
    deg = dp_ref[:, 0:1] + 1.0
    return lax.rsqrt(jnp.maximum(deg, 1e-12))


def _tc1(x, W1, degp3):
    def body(x_ref, w_ref, dp_ref, y_ref):
        i = pl.program_id(0)

        @pl.when(i < NB)
        def _():
            dinv = _dinv_block(dp_ref)
            y_ref[...] = jnp.dot(x_ref[...], w_ref[...],
                                 preferred_element_type=_f32) * dinv

        @pl.when(i == NB)
        def _():
            y_ref[...] = jnp.zeros_like(y_ref)

    clamp = lambda i: (jnp.minimum(i, NB - 1), 0)
    return pl.pallas_call(
        body,
        grid=(NB + 1,),
        in_specs=[
            pl.BlockSpec((BN, 128), clamp),
            pl.BlockSpec((128, 128), lambda i: (0, 0)),
            pl.BlockSpec((BN, 128), clamp),
        ],
        out_specs=pl.BlockSpec((BN, 128), lambda i: (i, 0)),
        out_shape=jax.ShapeDtypeStruct((N + 1000, 128), _f32),
    )(x, W1, degp3)


def _tc2(s1, y1, degp3, b1r, W2, Wres, bresr):
    def body(s_ref, y_ref, dp_ref, b1_ref, w2_ref, wr_ref, br_ref, y2_ref, r_ref):
        i = pl.program_id(0)

        @pl.when(i < NB)
        def _():
            dinv = _dinv_block(dp_ref)
            h1 = jax.nn.relu((s_ref[...] + y_ref[...]) * dinv + b1_ref[...])
            m2 = jnp.dot(h1, w2_ref[...], preferred_element_type=_f32) * dinv
            y2_ref[0, :, :] = m2[:, :128]
            y2_ref[1, :, :] = m2[:, 128:]
            r_ref[...] = jnp.dot(h1, wr_ref[...],
                                 preferred_element_type=_f32) + br_ref[...]

        @pl.when(i == NB)
        def _():
            y2_ref[...] = jnp.zeros_like(y2_ref)

    clamp = lambda i: (jnp.minimum(i, NB - 1), 0)
    return pl.pallas_call(
        body,
        grid=(NB + 1,),
        in_specs=[
            pl.BlockSpec((BN, 128), clamp),
            pl.BlockSpec((BN, 128), clamp),
            pl.BlockSpec((BN, 128), clamp),
            pl.BlockSpec((1, 128), lambda i: (0, 0)),
            pl.BlockSpec((128, 256), lambda i: (0, 0)),
            pl.BlockSpec((128, 256), lambda i: (0, 0)),
            pl.BlockSpec((1, 256), lambda i: (0, 0)),
        ],
        out_specs=[
            pl.BlockSpec((2, BN, 128), lambda i: (0, i, 0)),
            pl.BlockSpec((BN, 256), clamp),
        ],
        out_shape=[
            jax.ShapeDtypeStruct((2, T2H, 128), _f32),
            jax.ShapeDtypeStruct((N, 256), _f32),
        ],
    )(s1, y1, degp3, b1r, W2, Wres, bresr)


def _tc3(s2_3, y2s, r, degp3, b2r, Wa1, ba1r, wa2r, ba2r, batch3):
    def body(s_ref, y_ref, r_ref, dp_ref, b2_ref, wa1_ref, ba1_ref,
             wa2_ref, ba2_ref, b_ref, out_ref, num_acc, cnt_acc):
        i = pl.program_id(0)
        dinv = _dinv_block(dp_ref)
        pre = jnp.concatenate([s_ref[0] + y_ref[0], s_ref[1] + y_ref[1]],
                              axis=1) * dinv
        h2 = jax.nn.relu(pre + b2_ref[...] + r_ref[...])
        t = jax.nn.relu(jnp.dot(h2, wa1_ref[...], preferred_element_type=_f32)
                        + ba1_ref[...])
        wl = jnp.sum(t * wa2_ref[...], axis=1, keepdims=True) + ba2_ref[...]
        w = jax.nn.sigmoid(wl)
        hw = h2 * w
        bb = b_ref[0]
        oneh = (bb == lax.broadcasted_iota(jnp.int32, (64, BN), 0)).astype(_f32)

        @pl.when(i == 0)
        def _():
            num_acc[...] = jnp.zeros_like(num_acc)
            cnt_acc[...] = jnp.zeros_like(cnt_acc)

        num_acc[...] = num_acc[...] + jnp.dot(oneh, hw,
                                              preferred_element_type=_f32)
        cnt_acc[...] = cnt_acc[...] + jnp.sum(oneh, axis=1, keepdims=True)

        @pl.when(i == NB - 1)
        def _():
            emb = num_acc[...] / jnp.maximum(cnt_acc[:, 0:1], 1.0)
            nrm2 = jnp.sum(emb * emb, axis=1, keepdims=True)
            out_ref[...] = emb / jnp.maximum(jnp.sqrt(nrm2), 1e-12)

    return pl.pallas_call(
        body,
        grid=(NB,),
        in_specs=[
            pl.BlockSpec((2, BN, 128), lambda i: (0, i, 0)),
            pl.BlockSpec((2, BN, 128), lambda i: (0, i, 0)),
            pl.BlockSpec((BN, 256), lambda i: (i, 0)),
            pl.BlockSpec((BN, 128), lambda i: (i, 0)),
            pl.BlockSpec((1, 256), lambda i: (0, 0)),
            pl.BlockSpec((256, 64), lambda i: (0, 0)),
            pl.BlockSpec((1, 64), lambda i: (0, 0)),
            pl.BlockSpec((1, 64), lambda i: (0, 0)),
            pl.BlockSpec((1, 1), lambda i: (0, 0)),
            pl.BlockSpec((1, 1, BN), lambda i: (i, 0, 0)),
        ],
        out_specs=pl.BlockSpec((64, 256), lambda i: (0, 0)),
        out_shape=jax.ShapeDtypeStruct((64, 256), _f32),
        scratch_shapes=[
            pltpu.VMEM((64, 256), _f32),
            pltpu.VMEM((64, 128), _f32),
        ],
    )(s2_3, y2s, r, degp3, b2r, Wa1, ba1r, wa2r, ba2r, batch3)


def kernel(x, edge_index, batch, W1, b1, W2, b2, Wres, bres, Wa1, ba1, Wa2, ba2):
    pad_i = jnp.arange(EPAD, dtype=jnp.int32)
    srcp = jnp.concatenate([edge_index[0], pad_i % N]).reshape(RR, K)
    dstp = jnp.concatenate([edge_index[1], NPAD + (pad_i % 64)]).reshape(RR, K)

    z128 = jnp.zeros((ACC, 128), _f32)

    ones_tbl = jnp.concatenate([jnp.ones((N, 128), _f32),
                                jnp.zeros((1000, 128), _f32)], axis=0)
    degp3 = _sc_agg1(ones_tbl, srcp, dstp, z128)

    y1 = _tc1(x, W1, degp3)
    s1 = _sc_agg1(y1, srcp, dstp, z128)
    y2s, r = _tc2(s1, y1, degp3, b1.reshape(1, 128),
                  W2, Wres, bres.reshape(1, 256))
    y2t = y2s.reshape(2 * T2H, 128)
    s2 = _sc_agg2(y2t, jnp.concatenate([srcp, srcp + T2H], axis=0), dstp, z128)
    return _tc3(s2.reshape(2, NPAD, 128), y2s, r, degp3, b2.reshape(1, 256),
                Wa1, ba1.reshape(1, 64), Wa2.reshape(1, 64),
                ba2.reshape(1, 1), batch.reshape(NB, 1, BN))


# speedup vs baseline: 9.9911x; 1.3161x over previous
import functools

import jax
import jax.numpy as jnp
from jax import lax
from jax.experimental import pallas as pl
from jax.experimental.pallas import tpu as pltpu
from jax.experimental.pallas import tpu_sc as plsc

N = 10000
E = 320000
K = 128
RR = 2560
EPAD = RR * K - E
CH = RR // 16
CH2 = CH // 4
NPAD = 10240
HN = NPAD // 2
ACC = HN
ZB1 = N
T2H = N + 1000
HPT = HN // 16
HACC = NPAD + 128
NPT = NPAD // 16
BN = 1000
NB = N // BN

_f32 = jnp.float32


def _sc_mesh():
    return plsc.VectorSubcoreMesh(core_axis_name="c", subcore_axis_name="s")


def _agg_phase(tblr, srcr, dstr, zr, srcv, dstv, rows, acc, sem, sem2, c, s,
               src_row0, zbase):
    lo = c * HN
    pltpu.sync_copy(zr.at[pl.ds(s * HPT, HPT)], acc.at[pl.ds(s * HPT, HPT)])
    plsc.subcore_barrier()

    def adjust(j, carry):
        for cc in range(0, K, 16):
            d = dstv[j, pl.ds(cc, 16)]
            inr = jnp.logical_and(d >= lo, d < lo + HN)
            dstv[j, pl.ds(cc, 16)] = jnp.where(inr, d - lo, d & 4095)
            sv = srcv[j, pl.ds(cc, 16)]
            srcv[j, pl.ds(cc, 16)] = jnp.where(inr, sv, zbase + (sv & 511))
        return carry

    for hb in range(CH // CH2):
        pltpu.sync_copy(srcr.at[pl.ds(src_row0 + s * CH + hb * CH2, CH2)], srcv)
        pltpu.sync_copy(dstr.at[pl.ds(s * CH + hb * CH2, CH2)], dstv)
        lax.fori_loop(0, CH2, adjust, 0)
        pltpu.async_copy(tblr.at[srcv.at[0]], rows[0], sem).wait()

        def body(j2, carry):
            j = 2 * j2
            for par in range(2):
                jn = jnp.minimum(j + par + 1, CH2 - 1)
                hs = pltpu.async_copy(rows[par], acc.at[dstv.at[j + par]],
                                      sem2, add=True)
                hg = pltpu.async_copy(tblr.at[srcv.at[jn]], rows[1 - par], sem)
                hg.wait()
                hs.wait()
            return carry

        lax.fori_loop(0, CH2 // 2, body, 0)
    plsc.subcore_barrier()


_AGG_SCRATCH = [
    pltpu.VMEM((CH2, K), jnp.int32),
    pltpu.VMEM((CH2, K), jnp.int32),
    pltpu.VMEM((K, 128), _f32),
    pltpu.VMEM((K, 128), _f32),
    pltpu.VMEM_SHARED((ACC, 128), _f32),
    pltpu.SemaphoreType.DMA,
    pltpu.SemaphoreType.DMA,
]


def _sc_agg1(tbl, srcarr, dstp, ztbl):
    @functools.partial(
        pl.kernel,
        out_type=jax.ShapeDtypeStruct((NPAD, 128), _f32),
        mesh=_sc_mesh(),
        scratch_types=list(_AGG_SCRATCH),
    )
    def k(tblr, srcr, dstr, zr, out, srcv, dstv, rows0, rows1, acc, sem, sem2):
        c = lax.axis_index("c")
        s = lax.axis_index("s")
        _agg_phase(tblr, srcr, dstr, zr, srcv, dstv, (rows0, rows1), acc,
                   sem, sem2, c, s, 0, ZB1)
        pltpu.sync_copy(acc.at[pl.ds(s * HPT, HPT)],
                        out.at[pl.ds(c * HN + s * HPT, HPT)])

    return k(tbl, srcarr, dstp, ztbl)


def _sc_agg2(tbl, src2, dstp, ztbl):
    @functools.partial(
        pl.kernel,
        out_type=jax.ShapeDtypeStruct((2 * NPAD, 128), _f32),
        mesh=_sc_mesh(),
        scratch_types=list(_AGG_SCRATCH),
    )
    def k(tblr, srcr, dstr, zr, out, srcv, dstv, rows0, rows1, acc, sem, sem2):
        c = lax.axis_index("c")
        s = lax.axis_index("s")
        for p in range(2):
            _agg_phase(tblr, srcr, dstr, zr, srcv, dstv, (rows0, rows1), acc,
                       sem, sem2, c, s, p * RR, N + p * T2H)
            pltpu.sync_copy(acc.at[pl.ds(s * HPT, HPT)],
                            out.at[pl.ds(p * NPAD + c * HN + s * HPT, HPT)])
            plsc.subcore_barrier()

    return k(tbl, src2, dstp, ztbl)


def _dinv_block(dp_ref):
    deg = dp_ref[:, 0:1] + 1.0
    return lax.rsqrt(jnp.maximum(deg, 1e-12))


def _tc1(x, W1, degp3):
    def body(x_ref, w_ref, dp_ref, y_ref):
        i = pl.program_id(0)

        @pl.when(i < NB)
        def _():
            dinv = _dinv_block(dp_ref)
            y_ref[...] = jnp.dot(x_ref[...], w_ref[...],
                                 preferred_element_type=_f32) * dinv

        @pl.when(i == NB)
        def _():
            y_ref[...] = jnp.zeros_like(y_ref)

    clamp = lambda i: (jnp.minimum(i, NB - 1), 0)
    return pl.pallas_call(
        body,
        grid=(NB + 1,),
        in_specs=[
            pl.BlockSpec((BN, 128), clamp),
            pl.BlockSpec((128, 128), lambda i: (0, 0)),
            pl.BlockSpec((BN, 128), clamp),
        ],
        out_specs=pl.BlockSpec((BN, 128), lambda i: (i, 0)),
        out_shape=jax.ShapeDtypeStruct((N + 1000, 128), _f32),
    )(x, W1, degp3)


def _tc2(s1, y1, degp3, b1r, W2, Wres, bresr):
    def body(s_ref, y_ref, dp_ref, b1_ref, w2_ref, wr_ref, br_ref, y2_ref, r_ref):
        i = pl.program_id(0)

        @pl.when(i < NB)
        def _():
            dinv = _dinv_block(dp_ref)
            h1 = jax.nn.relu((s_ref[...] + y_ref[...]) * dinv + b1_ref[...])
            m2 = jnp.dot(h1, w2_ref[...], preferred_element_type=_f32) * dinv
            y2_ref[0, :, :] = m2[:, :128]
            y2_ref[1, :, :] = m2[:, 128:]
            r_ref[...] = jnp.dot(h1, wr_ref[...],
                                 preferred_element_type=_f32) + br_ref[...]

        @pl.when(i == NB)
        def _():
            y2_ref[...] = jnp.zeros_like(y2_ref)

    clamp = lambda i: (jnp.minimum(i, NB - 1), 0)
    return pl.pallas_call(
        body,
        grid=(NB + 1,),
        in_specs=[
            pl.BlockSpec((BN, 128), clamp),
            pl.BlockSpec((BN, 128), clamp),
            pl.BlockSpec((BN, 128), clamp),
            pl.BlockSpec((1, 128), lambda i: (0, 0)),
            pl.BlockSpec((128, 256), lambda i: (0, 0)),
            pl.BlockSpec((128, 256), lambda i: (0, 0)),
            pl.BlockSpec((1, 256), lambda i: (0, 0)),
        ],
        out_specs=[
            pl.BlockSpec((2, BN, 128), lambda i: (0, i, 0)),
            pl.BlockSpec((BN, 256), clamp),
        ],
        out_shape=[
            jax.ShapeDtypeStruct((2, T2H, 128), _f32),
            jax.ShapeDtypeStruct((N, 256), _f32),
        ],
    )(s1, y1, degp3, b1r, W2, Wres, bresr)


def _tc3(s2_3, y2s, r, degp3, b2r, Wa1, ba1r, wa2r, ba2r, batch3):
    def body(s_ref, y_ref, r_ref, dp_ref, b2_ref, wa1_ref, ba1_ref,
             wa2_ref, ba2_ref, b_ref, out_ref, num_acc, cnt_acc):
        i = pl.program_id(0)
        dinv = _dinv_block(dp_ref)
        pre = jnp.concatenate([s_ref[0] + y_ref[0], s_ref[1] + y_ref[1]],
                              axis=1) * dinv
        h2 = jax.nn.relu(pre + b2_ref[...] + r_ref[...])
        t = jax.nn.relu(jnp.dot(h2, wa1_ref[...], preferred_element_type=_f32)
                        + ba1_ref[...])
        wl = jnp.sum(t * wa2_ref[...], axis=1, keepdims=True) + ba2_ref[...]
        w = jax.nn.sigmoid(wl)
        hw = h2 * w
        bb = b_ref[0]
        oneh = (bb == lax.broadcasted_iota(jnp.int32, (64, BN), 0)).astype(_f32)

        @pl.when(i == 0)
        def _():
            num_acc[...] = jnp.zeros_like(num_acc)
            cnt_acc[...] = jnp.zeros_like(cnt_acc)

        num_acc[...] = num_acc[...] + jnp.dot(oneh, hw,
                                              preferred_element_type=_f32)
        cnt_acc[...] = cnt_acc[...] + jnp.sum(oneh, axis=1, keepdims=True)

        @pl.when(i == NB - 1)
        def _():
            emb = num_acc[...] / jnp.maximum(cnt_acc[:, 0:1], 1.0)
            nrm2 = jnp.sum(emb * emb, axis=1, keepdims=True)
            out_ref[...] = emb / jnp.maximum(jnp.sqrt(nrm2), 1e-12)

    return pl.pallas_call(
        body,
        grid=(NB,),
        in_specs=[
            pl.BlockSpec((2, BN, 128), lambda i: (0, i, 0)),
            pl.BlockSpec((2, BN, 128), lambda i: (0, i, 0)),
            pl.BlockSpec((BN, 256), lambda i: (i, 0)),
            pl.BlockSpec((BN, 128), lambda i: (i, 0)),
            pl.BlockSpec((1, 256), lambda i: (0, 0)),
            pl.BlockSpec((256, 64), lambda i: (0, 0)),
            pl.BlockSpec((1, 64), lambda i: (0, 0)),
            pl.BlockSpec((1, 64), lambda i: (0, 0)),
            pl.BlockSpec((1, 1), lambda i: (0, 0)),
            pl.BlockSpec((1, 1, BN), lambda i: (i, 0, 0)),
        ],
        out_specs=pl.BlockSpec((64, 256), lambda i: (0, 0)),
        out_shape=jax.ShapeDtypeStruct((64, 256), _f32),
        scratch_shapes=[
            pltpu.VMEM((64, 256), _f32),
            pltpu.VMEM((64, 128), _f32),
        ],
    )(s2_3, y2s, r, degp3, b2r, Wa1, ba1r, wa2r, ba2r, batch3)


def kernel(x, edge_index, batch, W1, b1, W2, b2, Wres, bres, Wa1, ba1, Wa2, ba2):
    pad_i = jnp.arange(EPAD, dtype=jnp.int32)
    srcp = jnp.concatenate([edge_index[0], pad_i % N]).reshape(RR, K)
    dstp = jnp.concatenate([edge_index[1], NPAD + (pad_i % 64)]).reshape(RR, K)

    z128 = jnp.zeros((ACC, 128), _f32)

    ones_tbl = jnp.concatenate([jnp.ones((N, 128), _f32),
                                jnp.zeros((1000, 128), _f32)], axis=0)
    degp3 = _sc_agg1(ones_tbl, srcp, dstp, z128)

    y1 = _tc1(x, W1, degp3)
    s1 = _sc_agg1(y1, srcp, dstp, z128)
    y2s, r = _tc2(s1, y1, degp3, b1.reshape(1, 128),
                  W2, Wres, bres.reshape(1, 256))
    y2t = y2s.reshape(2 * T2H, 128)
    s2 = _sc_agg2(y2t, jnp.concatenate([srcp, srcp + T2H], axis=0), dstp, z128)
    return _tc3(s2.reshape(2, NPAD, 128), y2s, r, degp3, b2.reshape(1, 256),
                Wa1, ba1.reshape(1, 64), Wa2.reshape(1, 64),
                ba2.reshape(1, 1), batch.reshape(NB, 1, BN))
